# baseline (speedup 1.0000x reference)
import functools

import jax
import jax.numpy as jnp
from jax.experimental import pallas as pl
import jax.experimental.pallas.tpu as pltpu

MARGIN_C = 0.05
N_MEANS = 65536
N_BATCH = 16
N_PTS = 1024
KT = 4096
KS = 8
UNROLL = 8
BIG = 3.0e38


def _rbf(x):
    return x.astype(jnp.bfloat16).astype(jnp.float32)


def _nn_loss_kernel(p_ref, m_ref, loss_ref, dmin_ref, nnx_ref, nny_ref,
                    nnz_ref, *, n_k_tiles):
    b = pl.program_id(0)
    k = pl.program_id(1)

    def sc(r, c):
        return p_ref[0, r:r + 1, c:c + 1]

    ox = _rbf(p_ref[0, 0:1, :])
    oy = _rbf(p_ref[0, 1:2, :])
    oz = _rbf(p_ref[0, 2:3, :])
    qx = ox * _rbf(sc(3, 0)) + oy * _rbf(sc(3, 1)) + oz * _rbf(sc(3, 2)) + sc(6, 0)
    qy = ox * _rbf(sc(4, 0)) + oy * _rbf(sc(4, 1)) + oz * _rbf(sc(4, 2)) + sc(6, 1)
    qz = ox * _rbf(sc(5, 0)) + oy * _rbf(sc(5, 1)) + oz * _rbf(sc(5, 2)) + sc(6, 2)
    qbx, qby, qbz = _rbf(qx), _rbf(qy), _rbf(qz)
    qsq = qx * qx + qy * qy + qz * qz

    @pl.when(k == 0)
    def _init():
        dmin_ref[...] = jnp.full((KS, N_PTS), BIG, jnp.float32)
        nnx_ref[...] = jnp.zeros((KS, N_PTS), jnp.float32)
        nny_ref[...] = jnp.zeros((KS, N_PTS), jnp.float32)
        nnz_ref[...] = jnp.zeros((KS, N_PTS), jnp.float32)

    def step(i, carry):
        dmin, nnx, nny, nnz = carry
        for u in range(UNROLL):
            base = i * (KS * UNROLL) + u * KS
            m = m_ref[pl.ds(base, KS), :]
            mx = m[:, 0:1]
            my = m[:, 1:2]
            mz = m[:, 2:3]
            msq = mx * mx + my * my + mz * mz
            mbx, mby, mbz = _rbf(mx), _rbf(my), _rbf(mz)
            dot = mbx * qbx + mby * qby + mbz * qbz
            d = (qsq + msq) - 2.0 * dot
            take = d < dmin
            dmin = jnp.where(take, d, dmin)
            nnx = jnp.where(take, mx, nnx)
            nny = jnp.where(take, my, nny)
            nnz = jnp.where(take, mz, nnz)
        return dmin, nnx, nny, nnz

    carry0 = (dmin_ref[...], nnx_ref[...], nny_ref[...], nnz_ref[...])
    dmin, nnx, nny, nnz = jax.lax.fori_loop(
        0, KT // (KS * UNROLL), step, carry0)
    dmin_ref[...] = dmin
    nnx_ref[...] = nnx
    nny_ref[...] = nny
    nnz_ref[...] = nnz

    @pl.when(k == n_k_tiles - 1)
    def _finish():
        db = dmin[0:1, :]
        xb = nnx[0:1, :]
        yb = nny[0:1, :]
        zb = nnz[0:1, :]
        for s in range(1, KS):
            cond = dmin[s:s + 1, :] < db
            db = jnp.where(cond, dmin[s:s + 1, :], db)
            xb = jnp.where(cond, nnx[s:s + 1, :], xb)
            yb = jnp.where(cond, nny[s:s + 1, :], yb)
            zb = jnp.where(cond, nnz[s:s + 1, :], zb)
        dx = qx - xb
        dy = qy - yb
        dz = qz - zb
        d2 = dx * dx + dy * dy + dz * dz
        contrib = jnp.sum(jnp.maximum(MARGIN_C - d2, 0.0)) / float(
            N_BATCH * N_PTS)

        @pl.when(b == 0)
        def _first():
            loss_ref[...] = jnp.zeros((1, 1), jnp.float32)

        loss_ref[...] = loss_ref[...] + contrib.reshape(1, 1)


@jax.jit
def kernel(outputs, c2ws, scene_scales, means):
    aff = c2ws[:, :3, :3] * scene_scales[:, None, None]
    trans = c2ws[:, :3, 3]
    pts = jnp.transpose(outputs, (0, 2, 1))
    aff_rows = jnp.pad(aff, ((0, 0), (0, 0), (0, N_PTS - 3)))
    trans_row = jnp.pad(trans[:, None, :], ((0, 0), (0, 0), (0, N_PTS - 3)))
    zero_row = jnp.zeros((N_BATCH, 1, N_PTS), jnp.float32)
    packed = jnp.concatenate([pts, aff_rows, trans_row, zero_row], axis=1)

    n_k_tiles = N_MEANS // KT
    loss = pl.pallas_call(
        functools.partial(_nn_loss_kernel, n_k_tiles=n_k_tiles),
        grid=(N_BATCH, n_k_tiles),
        in_specs=[
            pl.BlockSpec((1, 8, N_PTS), lambda b, k: (b, 0, 0)),
            pl.BlockSpec((KT, 3), lambda b, k: (k, 0)),
        ],
        out_specs=pl.BlockSpec((1, 1), lambda b, k: (0, 0)),
        out_shape=jax.ShapeDtypeStruct((1, 1), jnp.float32),
        scratch_shapes=[
            pltpu.VMEM((KS, N_PTS), jnp.float32),
            pltpu.VMEM((KS, N_PTS), jnp.float32),
            pltpu.VMEM((KS, N_PTS), jnp.float32),
            pltpu.VMEM((KS, N_PTS), jnp.float32),
        ],
        compiler_params=pltpu.CompilerParams(
            dimension_semantics=("arbitrary", "arbitrary")),
    )(packed, means)
    return loss[0, 0]


# speedup vs baseline: 3.8236x; 3.8236x over previous
import functools

import jax
import jax.numpy as jnp
from jax import lax
from jax.experimental import pallas as pl
from jax.experimental.pallas import tpu as pltpu
from jax.experimental.pallas import tpu_sc as plsc

MARGIN_C = 0.05
N_MEANS = 65536
N_BATCH = 16
N_PTS = 1024
N_Q = N_BATCH * N_PTS
KT = 4096
W = 128
BIG = 3.0e38
IDX_BIG = 1.6e7


def _tc_kernel(p_ref, a_ref, m_ref, idx_ref, q_ref, rund_ref, runi_ref,
               *, n_k_tiles):
    k = pl.program_id(1)

    pts = p_ref[0]
    A = a_ref[0]
    qfull = jnp.dot(pts.astype(jnp.bfloat16), A.astype(jnp.bfloat16),
                    preferred_element_type=jnp.float32)
    q8 = qfull[:, 0:8] + A[3:4, 0:8]
    qb8 = q8.astype(jnp.bfloat16)

    @pl.when(k == 0)
    def _init():
        rund_ref[...] = jnp.full((N_PTS, W), BIG, jnp.float32)
        runi_ref[...] = jnp.zeros((N_PTS, W), jnp.float32)

    iota_l = lax.broadcasted_iota(jnp.int32, (1, W), 1).astype(jnp.float32)
    for c in range(KT // W):
        mb = m_ref[:, pl.ds(c * W, W)].astype(jnp.bfloat16)
        dot = jnp.dot(qb8, mb, preferred_element_type=jnp.float32)
        msq = m_ref[3:4, pl.ds(c * W, W)]
        t = msq - 2.0 * dot
        idx_c = iota_l + float(c * W) + lax.convert_element_type(
            k * KT, jnp.float32)
        take = t < rund_ref[...]
        rund_ref[...] = jnp.where(take, t, rund_ref[...])
        runi_ref[...] = jnp.where(take, idx_c, runi_ref[...])

    @pl.when(k == n_k_tiles - 1)
    def _finish():
        rund = rund_ref[...]
        runi = runi_ref[...]
        mind = jnp.min(rund, axis=1, keepdims=True)
        cand = jnp.where(rund == mind, runi, IDX_BIG)
        idxq = jnp.min(cand, axis=1, keepdims=True)
        idx_ref[0] = jnp.broadcast_to(idxq, (N_PTS, 8))
        q_ref[0] = q8


def _sc_stage(mx_h, my_h, mz_h, idx_i32, qx, qy, qz):
    info = plsc.get_sparse_core_info()
    nc, ns = info.num_cores, info.num_subcores
    nw = nc * ns
    bpw = N_Q // nw
    mesh = plsc.VectorSubcoreMesh(core_axis_name="c", subcore_axis_name="s")

    @functools.partial(
        pl.kernel, mesh=mesh,
        out_type=jax.ShapeDtypeStruct((nw, 16), jnp.float32),
        scratch_types=[
            pltpu.VMEM((bpw,), jnp.int32),
            pltpu.VMEM((bpw,), jnp.float32),
            pltpu.VMEM((bpw,), jnp.float32),
            pltpu.VMEM((bpw,), jnp.float32),
            pltpu.VMEM((bpw,), jnp.float32),
            pltpu.VMEM((bpw,), jnp.float32),
            pltpu.VMEM((bpw,), jnp.float32),
            pltpu.VMEM((16,), jnp.float32),
            pltpu.SemaphoreType.DMA,
            pltpu.SemaphoreType.DMA,
            pltpu.SemaphoreType.DMA,
        ],
    )
    def sc_body(mx_hbm, my_hbm, mz_hbm, idx_hbm, qx_hbm, qy_hbm, qz_hbm,
                out_hbm, idx_v, mx_v, my_v, mz_v, qx_v, qy_v, qz_v, acc_v,
                sem0, sem1, sem2):
        wid = lax.axis_index("s") * nc + lax.axis_index("c")
        base = wid * bpw
        pltpu.sync_copy(idx_hbm.at[pl.ds(base, bpw)], idx_v)
        cx = pltpu.async_copy(mx_hbm.at[idx_v], mx_v, sem0)
        cy = pltpu.async_copy(my_hbm.at[idx_v], my_v, sem1)
        cz = pltpu.async_copy(mz_hbm.at[idx_v], mz_v, sem2)
        pltpu.sync_copy(qx_hbm.at[pl.ds(base, bpw)], qx_v)
        pltpu.sync_copy(qy_hbm.at[pl.ds(base, bpw)], qy_v)
        pltpu.sync_copy(qz_hbm.at[pl.ds(base, bpw)], qz_v)
        cx.wait()
        cy.wait()
        cz.wait()

        acc = jnp.zeros((16,), jnp.float32)
        for i in range(bpw // 16):
            sl = pl.ds(i * 16, 16)
            dx = qx_v[sl] - mx_v[sl]
            dy = qy_v[sl] - my_v[sl]
            dz = qz_v[sl] - mz_v[sl]
            d2 = (dx * dx + dy * dy) + dz * dz
            acc = acc + jnp.maximum(MARGIN_C - d2, 0.0)
        acc_v[...] = acc
        pltpu.sync_copy(acc_v, out_hbm.at[wid])

    return sc_body(mx_h, my_h, mz_h, idx_i32, qx, qy, qz)


@jax.jit
def kernel(outputs, c2ws, scene_scales, means):
    aff = c2ws[:, :3, :3] * scene_scales[:, None, None]
    trans = c2ws[:, :3, 3]
    pts8 = jnp.pad(outputs, ((0, 0), (0, 0), (0, 5)))
    affT = jnp.transpose(aff, (0, 2, 1))
    abar = jnp.zeros((N_BATCH, 8, 128), jnp.float32)
    abar = abar.at[:, 0:3, 0:3].set(affT)
    abar = abar.at[:, 3, 0:3].set(trans)
    msq = jnp.sum(means * means, axis=1)
    mt = jnp.concatenate(
        [means.T, msq[None, :], jnp.zeros((4, N_MEANS), jnp.float32)],
        axis=0)

    n_k_tiles = N_MEANS // KT
    idxs, qs = pl.pallas_call(
        functools.partial(_tc_kernel, n_k_tiles=n_k_tiles),
        grid=(N_BATCH, n_k_tiles),
        in_specs=[
            pl.BlockSpec((1, N_PTS, 8), lambda b, k: (b, 0, 0)),
            pl.BlockSpec((1, 8, 128), lambda b, k: (b, 0, 0)),
            pl.BlockSpec((8, KT), lambda b, k: (0, k)),
        ],
        out_specs=[
            pl.BlockSpec((1, N_PTS, 8), lambda b, k: (b, 0, 0)),
            pl.BlockSpec((1, N_PTS, 8), lambda b, k: (b, 0, 0)),
        ],
        out_shape=[
            jax.ShapeDtypeStruct((N_BATCH, N_PTS, 8), jnp.float32),
            jax.ShapeDtypeStruct((N_BATCH, N_PTS, 8), jnp.float32),
        ],
        scratch_shapes=[
            pltpu.VMEM((N_PTS, W), jnp.float32),
            pltpu.VMEM((N_PTS, W), jnp.float32),
        ],
        compiler_params=pltpu.CompilerParams(
            dimension_semantics=("arbitrary", "arbitrary")),
    )(pts8, abar, mt)

    idx_i32 = idxs[:, :, 0].reshape(N_Q).astype(jnp.int32)
    qx = qs[:, :, 0].reshape(N_Q)
    qy = qs[:, :, 1].reshape(N_Q)
    qz = qs[:, :, 2].reshape(N_Q)
    mx_h = means[:, 0]
    my_h = means[:, 1]
    mz_h = means[:, 2]

    partials = _sc_stage(mx_h, my_h, mz_h, idx_i32, qx, qy, qz)
    return jnp.sum(partials) / float(N_Q)


# speedup vs baseline: 4.2557x; 1.1130x over previous
import functools

import jax
import jax.numpy as jnp
from jax import lax
from jax.experimental import pallas as pl
from jax.experimental.pallas import tpu as pltpu
from jax.experimental.pallas import tpu_sc as plsc

MARGIN_C = 0.05
N_MEANS = 65536
N_BATCH = 16
N_PTS = 1024
N_Q = N_BATCH * N_PTS
KT = 4096
W = 128
BIG = 3.0e38
IDX_BIG = 1.6e7


def _tc_kernel(p_ref, a_ref, m_ref, idx_ref, q_ref, rund_ref, runi_ref,
               *, n_k_tiles):
    k = pl.program_id(1)

    pts = p_ref[0]
    A = a_ref[0]
    qfull = jnp.dot(pts.astype(jnp.bfloat16), A.astype(jnp.bfloat16),
                    preferred_element_type=jnp.float32)
    q8 = qfull[:, 0:8] + A[3:4, 0:8]
    qb8 = q8.astype(jnp.bfloat16)

    @pl.when(k == 0)
    def _init():
        rund_ref[...] = jnp.full((N_PTS, W), BIG, jnp.float32)
        runi_ref[...] = jnp.zeros((N_PTS, W), jnp.float32)

    iota_l = lax.broadcasted_iota(jnp.int32, (1, W), 1).astype(jnp.float32)
    rund = rund_ref[...]
    runi = runi_ref[...]
    for c in range(KT // W):
        mb = m_ref[:, pl.ds(c * W, W)].astype(jnp.bfloat16)
        dot = jnp.dot(qb8, mb, preferred_element_type=jnp.float32)
        t = m_ref[3:4, pl.ds(c * W, W)] + dot
        idx_c = iota_l + float(c * W) + lax.convert_element_type(
            k * KT, jnp.float32)
        take = t < rund
        rund = jnp.where(take, t, rund)
        runi = jnp.where(take, idx_c, runi)
    rund_ref[...] = rund
    runi_ref[...] = runi

    @pl.when(k == n_k_tiles - 1)
    def _finish():
        mind = jnp.min(rund, axis=1, keepdims=True)
        cand = jnp.where(rund == mind, runi, IDX_BIG)
        idxq = jnp.min(cand, axis=1, keepdims=True)
        idx_ref[0] = jnp.broadcast_to(idxq, (N_PTS, 8))
        q_ref[0] = q8


def _sc_stage(mx_h, my_h, mz_h, idx_i32, qx, qy, qz):
    info = plsc.get_sparse_core_info()
    nc, ns = info.num_cores, info.num_subcores
    nw = nc * ns
    bpw = N_Q // nw
    mesh = plsc.VectorSubcoreMesh(core_axis_name="c", subcore_axis_name="s")

    @functools.partial(
        pl.kernel, mesh=mesh,
        out_type=jax.ShapeDtypeStruct((nw, 16), jnp.float32),
        scratch_types=[
            pltpu.VMEM((bpw,), jnp.int32),
            pltpu.VMEM((bpw,), jnp.float32),
            pltpu.VMEM((bpw,), jnp.float32),
            pltpu.VMEM((bpw,), jnp.float32),
            pltpu.VMEM((bpw,), jnp.float32),
            pltpu.VMEM((bpw,), jnp.float32),
            pltpu.VMEM((bpw,), jnp.float32),
            pltpu.VMEM((16,), jnp.float32),
            pltpu.SemaphoreType.DMA,
            pltpu.SemaphoreType.DMA,
            pltpu.SemaphoreType.DMA,
        ],
    )
    def sc_body(mx_hbm, my_hbm, mz_hbm, idx_hbm, qx_hbm, qy_hbm, qz_hbm,
                out_hbm, idx_v, mx_v, my_v, mz_v, qx_v, qy_v, qz_v, acc_v,
                sem0, sem1, sem2):
        wid = lax.axis_index("s") * nc + lax.axis_index("c")
        base = wid * bpw
        pltpu.sync_copy(idx_hbm.at[pl.ds(base, bpw)], idx_v)
        cx = pltpu.async_copy(mx_hbm.at[idx_v], mx_v, sem0)
        cy = pltpu.async_copy(my_hbm.at[idx_v], my_v, sem1)
        cz = pltpu.async_copy(mz_hbm.at[idx_v], mz_v, sem2)
        pltpu.sync_copy(qx_hbm.at[pl.ds(base, bpw)], qx_v)
        pltpu.sync_copy(qy_hbm.at[pl.ds(base, bpw)], qy_v)
        pltpu.sync_copy(qz_hbm.at[pl.ds(base, bpw)], qz_v)
        cx.wait()
        cy.wait()
        cz.wait()

        acc = jnp.zeros((16,), jnp.float32)
        for i in range(bpw // 16):
            sl = pl.ds(i * 16, 16)
            dx = qx_v[sl] - mx_v[sl]
            dy = qy_v[sl] - my_v[sl]
            dz = qz_v[sl] - mz_v[sl]
            d2 = (dx * dx + dy * dy) + dz * dz
            acc = acc + jnp.maximum(MARGIN_C - d2, 0.0)
        acc_v[...] = acc
        pltpu.sync_copy(acc_v, out_hbm.at[wid])

    return sc_body(mx_h, my_h, mz_h, idx_i32, qx, qy, qz)


@jax.jit
def kernel(outputs, c2ws, scene_scales, means):
    aff = c2ws[:, :3, :3] * scene_scales[:, None, None]
    trans = c2ws[:, :3, 3]
    pts8 = jnp.pad(outputs, ((0, 0), (0, 0), (0, 5)))
    affT = jnp.transpose(aff, (0, 2, 1))
    abar = jnp.zeros((N_BATCH, 8, 128), jnp.float32)
    abar = abar.at[:, 0:3, 0:3].set(affT)
    abar = abar.at[:, 3, 0:3].set(trans)
    msq = jnp.sum(means * means, axis=1)
    mt = jnp.concatenate(
        [-2.0 * means.T, msq[None, :], jnp.zeros((4, N_MEANS), jnp.float32)],
        axis=0)

    n_k_tiles = N_MEANS // KT
    idxs, qs = pl.pallas_call(
        functools.partial(_tc_kernel, n_k_tiles=n_k_tiles),
        grid=(N_BATCH, n_k_tiles),
        in_specs=[
            pl.BlockSpec((1, N_PTS, 8), lambda b, k: (b, 0, 0)),
            pl.BlockSpec((1, 8, 128), lambda b, k: (b, 0, 0)),
            pl.BlockSpec((8, KT), lambda b, k: (0, k)),
        ],
        out_specs=[
            pl.BlockSpec((1, N_PTS, 8), lambda b, k: (b, 0, 0)),
            pl.BlockSpec((1, N_PTS, 8), lambda b, k: (b, 0, 0)),
        ],
        out_shape=[
            jax.ShapeDtypeStruct((N_BATCH, N_PTS, 8), jnp.float32),
            jax.ShapeDtypeStruct((N_BATCH, N_PTS, 8), jnp.float32),
        ],
        scratch_shapes=[
            pltpu.VMEM((N_PTS, W), jnp.float32),
            pltpu.VMEM((N_PTS, W), jnp.float32),
        ],
        compiler_params=pltpu.CompilerParams(
            dimension_semantics=("arbitrary", "arbitrary")),
    )(pts8, abar, mt)

    idx_i32 = idxs[:, :, 0].reshape(N_Q).astype(jnp.int32)
    qx = qs[:, :, 0].reshape(N_Q)
    qy = qs[:, :, 1].reshape(N_Q)
    qz = qs[:, :, 2].reshape(N_Q)
    mx_h = means[:, 0]
    my_h = means[:, 1]
    mz_h = means[:, 2]

    partials = _sc_stage(mx_h, my_h, mz_h, idx_i32, qx, qy, qz)
    return jnp.sum(partials) / float(N_Q)


# speedup vs baseline: 4.2639x; 1.0019x over previous
import functools

import jax
import jax.numpy as jnp
from jax import lax
from jax.experimental import pallas as pl
from jax.experimental.pallas import tpu as pltpu
from jax.experimental.pallas import tpu_sc as plsc

MARGIN_C = 0.05
N_MEANS = 65536
N_BATCH = 16
N_PTS = 1024
N_Q = N_BATCH * N_PTS
KT = 4096
W = 128
BIG = 3.0e38
IDX_BIG = 1.6e7


def _tc_kernel(p_ref, a_ref, m_ref, idx_ref, q_ref, rund_ref, runi_ref,
               *, n_k_tiles):
    k = pl.program_id(1)

    pts = p_ref[0]
    A = a_ref[0]
    qfull = jnp.dot(pts.astype(jnp.bfloat16), A.astype(jnp.bfloat16),
                    preferred_element_type=jnp.float32)
    q8 = qfull[:, 0:8] + A[3:4, 0:8]
    qb8 = q8.astype(jnp.bfloat16)

    @pl.when(k == 0)
    def _init():
        rund_ref[...] = jnp.full((N_PTS, W), BIG, jnp.float32)
        runi_ref[...] = jnp.zeros((N_PTS, W), jnp.float32)

    iota_l = lax.broadcasted_iota(jnp.int32, (1, W), 1).astype(jnp.float32)
    rund = rund_ref[...]
    runi = runi_ref[...]
    for c in range(KT // W):
        mb = m_ref[:, pl.ds(c * W, W)].astype(jnp.bfloat16)
        dot = jnp.dot(qb8, mb, preferred_element_type=jnp.float32)
        t = m_ref[3:4, pl.ds(c * W, W)] + dot
        idx_c = iota_l + float(c * W) + lax.convert_element_type(
            k * KT, jnp.float32)
        take = t < rund
        rund = jnp.where(take, t, rund)
        runi = jnp.where(take, idx_c, runi)
    rund_ref[...] = rund
    runi_ref[...] = runi

    @pl.when(k == n_k_tiles - 1)
    def _finish():
        mind = jnp.min(rund, axis=1, keepdims=True)
        cand = jnp.where(rund == mind, runi, IDX_BIG)
        idxq = jnp.min(cand, axis=1, keepdims=True)
        idx_ref[0] = jnp.broadcast_to(idxq, (N_PTS, 8))
        q_ref[0] = q8


def _sc_stage(mx_h, my_h, mz_h, idx_i32, qx, qy, qz):
    info = plsc.get_sparse_core_info()
    nc, ns = info.num_cores, info.num_subcores
    nw = nc * ns
    bpw = N_Q // nw
    mesh = plsc.VectorSubcoreMesh(core_axis_name="c", subcore_axis_name="s")

    @functools.partial(
        pl.kernel, mesh=mesh,
        out_type=jax.ShapeDtypeStruct((nw, 16), jnp.float32),
        scratch_types=[
            pltpu.VMEM((bpw,), jnp.int32),
            pltpu.VMEM((bpw,), jnp.float32),
            pltpu.VMEM((bpw,), jnp.float32),
            pltpu.VMEM((bpw,), jnp.float32),
            pltpu.VMEM((bpw,), jnp.float32),
            pltpu.VMEM((bpw,), jnp.float32),
            pltpu.VMEM((bpw,), jnp.float32),
            pltpu.VMEM((16,), jnp.float32),
            pltpu.SemaphoreType.DMA,
            pltpu.SemaphoreType.DMA,
            pltpu.SemaphoreType.DMA,
        ],
    )
    def sc_body(mx_hbm, my_hbm, mz_hbm, idx_hbm, qx_hbm, qy_hbm, qz_hbm,
                out_hbm, idx_v, mx_v, my_v, mz_v, qx_v, qy_v, qz_v, acc_v,
                sem0, sem1, sem2):
        wid = lax.axis_index("s") * nc + lax.axis_index("c")
        base = wid * bpw
        pltpu.sync_copy(idx_hbm.at[pl.ds(base, bpw)], idx_v)
        cx = pltpu.async_copy(mx_hbm.at[idx_v], mx_v, sem0)
        cy = pltpu.async_copy(my_hbm.at[idx_v], my_v, sem1)
        cz = pltpu.async_copy(mz_hbm.at[idx_v], mz_v, sem2)
        pltpu.sync_copy(qx_hbm.at[pl.ds(base, bpw)], qx_v)
        pltpu.sync_copy(qy_hbm.at[pl.ds(base, bpw)], qy_v)
        pltpu.sync_copy(qz_hbm.at[pl.ds(base, bpw)], qz_v)
        cx.wait()
        cy.wait()
        cz.wait()

        acc = jnp.zeros((16,), jnp.float32)
        for i in range(bpw // 16):
            sl = pl.ds(i * 16, 16)
            dx = qx_v[sl] - mx_v[sl]
            dy = qy_v[sl] - my_v[sl]
            dz = qz_v[sl] - mz_v[sl]
            d2 = (dx * dx + dy * dy) + dz * dz
            acc = acc + jnp.maximum(MARGIN_C - d2, 0.0)
        acc_v[...] = acc
        pltpu.sync_copy(acc_v, out_hbm.at[wid])

    return sc_body(mx_h, my_h, mz_h, idx_i32, qx, qy, qz)


@jax.jit
def kernel(outputs, c2ws, scene_scales, means):
    aff = c2ws[:, :3, :3] * scene_scales[:, None, None]
    trans = c2ws[:, :3, 3]
    pts8 = jnp.pad(outputs, ((0, 0), (0, 0), (0, 5)))
    affT = jnp.transpose(aff, (0, 2, 1))
    abar = jnp.zeros((N_BATCH, 8, 128), jnp.float32)
    abar = abar.at[:, 0:3, 0:3].set(affT)
    abar = abar.at[:, 3, 0:3].set(trans)
    msq = jnp.sum(means * means, axis=1)
    mt = jnp.concatenate(
        [-2.0 * means.T, msq[None, :], jnp.zeros((4, N_MEANS), jnp.float32)],
        axis=0)

    n_k_tiles = N_MEANS // KT
    idxs, qs = pl.pallas_call(
        functools.partial(_tc_kernel, n_k_tiles=n_k_tiles),
        grid=(N_BATCH, n_k_tiles),
        in_specs=[
            pl.BlockSpec((1, N_PTS, 8), lambda b, k: (b, 0, 0)),
            pl.BlockSpec((1, 8, 128), lambda b, k: (b, 0, 0)),
            pl.BlockSpec((8, KT), lambda b, k: (0, k)),
        ],
        out_specs=[
            pl.BlockSpec((1, N_PTS, 8), lambda b, k: (b, 0, 0)),
            pl.BlockSpec((1, N_PTS, 8), lambda b, k: (b, 0, 0)),
        ],
        out_shape=[
            jax.ShapeDtypeStruct((N_BATCH, N_PTS, 8), jnp.float32),
            jax.ShapeDtypeStruct((N_BATCH, N_PTS, 8), jnp.float32),
        ],
        scratch_shapes=[
            pltpu.VMEM((N_PTS, W), jnp.float32),
            pltpu.VMEM((N_PTS, W), jnp.float32),
        ],
        compiler_params=pltpu.CompilerParams(
            dimension_semantics=("parallel", "arbitrary")),
    )(pts8, abar, mt)

    idx_i32 = idxs[:, :, 0].reshape(N_Q).astype(jnp.int32)
    qx = qs[:, :, 0].reshape(N_Q)
    qy = qs[:, :, 1].reshape(N_Q)
    qz = qs[:, :, 2].reshape(N_Q)
    mx_h = means[:, 0]
    my_h = means[:, 1]
    mz_h = means[:, 2]

    partials = _sc_stage(mx_h, my_h, mz_h, idx_i32, qx, qy, qz)
    return jnp.sum(partials) / float(N_Q)


# speedup vs baseline: 4.3226x; 1.0138x over previous
import functools

import jax
import jax.numpy as jnp
from jax import lax
from jax.experimental import pallas as pl
from jax.experimental.pallas import tpu as pltpu
from jax.experimental.pallas import tpu_sc as plsc

MARGIN_C = 0.05
N_MEANS = 65536
N_BATCH = 16
N_PTS = 1024
N_Q = N_BATCH * N_PTS
KT = 4096
W = 128
QC = 256
BIG = 3.0e38
IDX_BIG = 1.6e7


def _tc_kernel(p_ref, a_ref, m_ref, idx_ref, q_ref, rund_ref, runi_ref,
               *, n_k_tiles):
    k = pl.program_id(1)

    pts = p_ref[0]
    A = a_ref[0]
    qfull = jnp.dot(pts.astype(jnp.bfloat16), A.astype(jnp.bfloat16),
                    preferred_element_type=jnp.float32)
    q8 = qfull[:, 0:8] + A[3:4, 0:8]
    qb8 = q8.astype(jnp.bfloat16)

    @pl.when(k == 0)
    def _init():
        rund_ref[...] = jnp.full((N_PTS, W), BIG, jnp.float32)
        runi_ref[...] = jnp.zeros((N_PTS, W), jnp.float32)

    iota_l = lax.broadcasted_iota(jnp.int32, (1, W), 1).astype(jnp.float32)
    kbase = lax.convert_element_type(k * KT, jnp.float32)
    idx_cs = [iota_l + float(c * W) + kbase for c in range(KT // W)]
    mbs = [m_ref[:, pl.ds(c * W, W)].astype(jnp.bfloat16)
           for c in range(KT // W)]
    for qc in range(N_PTS // QC):
        qsl = pl.ds(qc * QC, QC)
        qbq = qb8[qc * QC:(qc + 1) * QC, :]
        rund = rund_ref[qsl, :]
        runi = runi_ref[qsl, :]
        for c in range(KT // W):
            dot = jnp.dot(qbq, mbs[c], preferred_element_type=jnp.float32)
            t = m_ref[3:4, pl.ds(c * W, W)] + dot
            take = t < rund
            rund = jnp.where(take, t, rund)
            runi = jnp.where(take, idx_cs[c], runi)
        rund_ref[qsl, :] = rund
        runi_ref[qsl, :] = runi

    @pl.when(k == n_k_tiles - 1)
    def _finish():
        rund = rund_ref[...]
        runi = runi_ref[...]
        mind = jnp.min(rund, axis=1, keepdims=True)
        cand = jnp.where(rund == mind, runi, IDX_BIG)
        idxq = jnp.min(cand, axis=1, keepdims=True)
        idx_ref[0] = jnp.broadcast_to(idxq, (N_PTS, 8))
        q_ref[0] = q8


def _sc_stage(mx_h, my_h, mz_h, idx_i32, qx, qy, qz):
    info = plsc.get_sparse_core_info()
    nc, ns = info.num_cores, info.num_subcores
    nw = nc * ns
    bpw = N_Q // nw
    mesh = plsc.VectorSubcoreMesh(core_axis_name="c", subcore_axis_name="s")

    @functools.partial(
        pl.kernel, mesh=mesh,
        out_type=jax.ShapeDtypeStruct((nw, 16), jnp.float32),
        scratch_types=[
            pltpu.VMEM((bpw,), jnp.int32),
            pltpu.VMEM((bpw,), jnp.float32),
            pltpu.VMEM((bpw,), jnp.float32),
            pltpu.VMEM((bpw,), jnp.float32),
            pltpu.VMEM((bpw,), jnp.float32),
            pltpu.VMEM((bpw,), jnp.float32),
            pltpu.VMEM((bpw,), jnp.float32),
            pltpu.VMEM((16,), jnp.float32),
            pltpu.SemaphoreType.DMA,
            pltpu.SemaphoreType.DMA,
            pltpu.SemaphoreType.DMA,
        ],
    )
    def sc_body(mx_hbm, my_hbm, mz_hbm, idx_hbm, qx_hbm, qy_hbm, qz_hbm,
                out_hbm, idx_v, mx_v, my_v, mz_v, qx_v, qy_v, qz_v, acc_v,
                sem0, sem1, sem2):
        wid = lax.axis_index("s") * nc + lax.axis_index("c")
        base = wid * bpw
        pltpu.sync_copy(idx_hbm.at[pl.ds(base, bpw)], idx_v)
        cx = pltpu.async_copy(mx_hbm.at[idx_v], mx_v, sem0)
        cy = pltpu.async_copy(my_hbm.at[idx_v], my_v, sem1)
        cz = pltpu.async_copy(mz_hbm.at[idx_v], mz_v, sem2)
        pltpu.sync_copy(qx_hbm.at[pl.ds(base, bpw)], qx_v)
        pltpu.sync_copy(qy_hbm.at[pl.ds(base, bpw)], qy_v)
        pltpu.sync_copy(qz_hbm.at[pl.ds(base, bpw)], qz_v)
        cx.wait()
        cy.wait()
        cz.wait()

        acc = jnp.zeros((16,), jnp.float32)
        for i in range(bpw // 16):
            sl = pl.ds(i * 16, 16)
            dx = qx_v[sl] - mx_v[sl]
            dy = qy_v[sl] - my_v[sl]
            dz = qz_v[sl] - mz_v[sl]
            d2 = (dx * dx + dy * dy) + dz * dz
            acc = acc + jnp.maximum(MARGIN_C - d2, 0.0)
        acc_v[...] = acc
        pltpu.sync_copy(acc_v, out_hbm.at[wid])

    return sc_body(mx_h, my_h, mz_h, idx_i32, qx, qy, qz)


@jax.jit
def kernel(outputs, c2ws, scene_scales, means):
    aff = c2ws[:, :3, :3] * scene_scales[:, None, None]
    trans = c2ws[:, :3, 3]
    pts8 = jnp.pad(outputs, ((0, 0), (0, 0), (0, 5)))
    affT = jnp.transpose(aff, (0, 2, 1))
    abar = jnp.zeros((N_BATCH, 8, 128), jnp.float32)
    abar = abar.at[:, 0:3, 0:3].set(affT)
    abar = abar.at[:, 3, 0:3].set(trans)
    msq = jnp.sum(means * means, axis=1)
    mt = jnp.concatenate(
        [-2.0 * means.T, msq[None, :], jnp.zeros((4, N_MEANS), jnp.float32)],
        axis=0)

    n_k_tiles = N_MEANS // KT
    idxs, qs = pl.pallas_call(
        functools.partial(_tc_kernel, n_k_tiles=n_k_tiles),
        grid=(N_BATCH, n_k_tiles),
        in_specs=[
            pl.BlockSpec((1, N_PTS, 8), lambda b, k: (b, 0, 0)),
            pl.BlockSpec((1, 8, 128), lambda b, k: (b, 0, 0)),
            pl.BlockSpec((8, KT), lambda b, k: (0, k)),
        ],
        out_specs=[
            pl.BlockSpec((1, N_PTS, 8), lambda b, k: (b, 0, 0)),
            pl.BlockSpec((1, N_PTS, 8), lambda b, k: (b, 0, 0)),
        ],
        out_shape=[
            jax.ShapeDtypeStruct((N_BATCH, N_PTS, 8), jnp.float32),
            jax.ShapeDtypeStruct((N_BATCH, N_PTS, 8), jnp.float32),
        ],
        scratch_shapes=[
            pltpu.VMEM((N_PTS, W), jnp.float32),
            pltpu.VMEM((N_PTS, W), jnp.float32),
        ],
        compiler_params=pltpu.CompilerParams(
            dimension_semantics=("parallel", "arbitrary")),
    )(pts8, abar, mt)

    idx_i32 = idxs[:, :, 0].reshape(N_Q).astype(jnp.int32)
    qx = qs[:, :, 0].reshape(N_Q)
    qy = qs[:, :, 1].reshape(N_Q)
    qz = qs[:, :, 2].reshape(N_Q)
    mx_h = means[:, 0]
    my_h = means[:, 1]
    mz_h = means[:, 2]

    partials = _sc_stage(mx_h, my_h, mz_h, idx_i32, qx, qy, qz)
    return jnp.sum(partials) / float(N_Q)
